# baseline (speedup 1.0000x reference)
import functools

import jax
import jax.numpy as jnp
from jax import lax
from jax.experimental import pallas as pl
from jax.experimental.pallas import tpu as pltpu
from jax.experimental.pallas import tpu_sc as plsc

K = 32
CH = 16
B, S, N, C = 8, 1024, 16384, 64
NCH = N // CH
NBLK = 1024
NW = 32
GROUPS = (B * S) // CH
GPW = GROUPS // NW
F32 = jnp.float32
I32 = jnp.int32


def _batcher_pairs(n):
    pairs = []
    p = 1
    while p < n:
        k = p
        while k >= 1:
            for j in range(k % p, n - k, 2 * k):
                for i in range(0, k):
                    if (i + j) // (2 * p) == (i + j + k) // (2 * p):
                        pairs.append((i + j, i + j + k))
            k //= 2
        p *= 2
    return pairs


NSEL = 40
_SORT64 = _batcher_pairs(64)


SBLK = 1024


def _tc_body(q_ref, x_ref, d_ref, m1_ref):
    q = q_ref[0]
    x = x_ref[0]
    qn = jnp.sum(q * q, axis=-1)
    xn = jnp.sum(x * x, axis=-1)
    ab = lax.dot_general(q, x, (((1,), (1,)), ((), ())),
                         preferred_element_type=F32)
    d = (-2.0 * ab + qn[:, None]) + xn[None, :]
    for j in range(NBLK // 128):
        d_ref[j, 0] = d[:, j * 128:(j + 1) * 128]
    ba = lax.dot_general(x, q, (((1,), (1,)), ((), ())),
                         preferred_element_type=F32)
    dt = (-2.0 * ba + qn[None, :]) + xn[:, None]
    m1t = jnp.min(dt.reshape(NBLK // CH, CH, SBLK), axis=1)
    for j in range(SBLK // 128):
        m1_ref[0, :, j] = m1t[:, j * 128:(j + 1) * 128]


def _tc_stage(xyz, new_xyz):
    grid = (B, N // NBLK)
    return pl.pallas_call(
        _tc_body,
        grid=grid,
        in_specs=[
            pl.BlockSpec((1, SBLK, C), lambda b, n: (b, 0, 0)),
            pl.BlockSpec((1, NBLK, C), lambda b, n: (b, n, 0)),
        ],
        out_specs=[
            pl.BlockSpec((NBLK // 128, 1, SBLK, 128),
                         lambda b, n: (0, n, b, 0)),
            pl.BlockSpec((1, NBLK // CH, S // 128, 128),
                         lambda b, n: (b, n, 0, 0)),
        ],
        out_shape=[
            jax.ShapeDtypeStruct((NBLK // 128, N // NBLK, B * S, 128), F32),
            jax.ShapeDtypeStruct((B, NCH, S // 128, 128), F32),
        ],
    )(new_xyz, xyz)


def _sc_body(d_hbm, m1_hbm, vals_hbm, idx_hbm,
             m1buf, l2, l3, cand, chunkids, coff, idxlist, r1, r2,
             valstage, idxstage, gsem):
    lanes = jnp.arange(16, dtype=I32)
    _INF = jnp.full((16,), jnp.inf, F32)
    wid = lax.axis_index("s") * 2 + lax.axis_index("c")

    def splat_i(v):
        return jnp.full((16,), v, I32)

    def group_body(g, carry):
        gid = wid * GPW + g
        row0 = gid * CH
        bb = gid // (S // CH)
        qq = (gid % (S // CH)) * CH
        pltpu.sync_copy(m1_hbm.at[pl.ds(bb * NCH, NCH), pl.ds(qq, 16)],
                        m1buf)

        def l2_body(j, c2):
            acc = _INF
            for t in range(CH):
                acc = jnp.minimum(
                    acc, plsc.load_gather(m1buf, [splat_i(j * 16 + t), lanes]))
            plsc.store_scatter(l2, [splat_i(j), lanes], acc)
            return c2

        lax.fori_loop(0, 64, l2_body, 0)
        for i in range(4):
            acc = _INF
            for t in range(CH):
                acc = jnp.minimum(acc, l2[16 * i + t])
            l3[i] = acc

        def p2_body(it, c2):
            v = jnp.minimum(jnp.minimum(l3[0], l3[1]),
                            jnp.minimum(l3[2], l3[3]))
            e3 = splat_i(3)
            for i in (2, 1, 0):
                e3 = jnp.where(l3[i] == v, splat_i(i), e3)
            base2 = e3 * 16
            e2 = base2
            for t in range(15, -1, -1):
                gvals = plsc.load_gather(l2, [base2 + t, lanes])
                e2 = jnp.where(gvals == v, base2 + t, e2)
            base1 = e2 * 16
            e1 = base1
            for t in range(15, -1, -1):
                gvals = plsc.load_gather(m1buf, [base1 + t, lanes])
                e1 = jnp.where(gvals == v, base1 + t, e1)
            plsc.store_scatter(chunkids, [splat_i(it), lanes], e1)
            plsc.store_scatter(m1buf, [e1, lanes], _INF)
            acc = _INF
            for t in range(CH):
                acc = jnp.minimum(acc, plsc.load_gather(m1buf, [base1 + t, lanes]))
            plsc.store_scatter(l2, [e2, lanes], acc)
            acc = _INF
            for t in range(CH):
                acc = jnp.minimum(acc, plsc.load_gather(l2, [e3 * 16 + t, lanes]))
            plsc.store_scatter(l3, [e3, lanes], acc)
            return c2

        lax.fori_loop(0, NSEL, p2_body, 0)

        vs = [chunkids[i] for i in range(NSEL)]
        vs += [jnp.full((16,), NCH, I32) for _ in range(64 - NSEL)]
        for (a, b) in _SORT64:
            lo = jnp.minimum(vs[a], vs[b])
            hi = jnp.maximum(vs[a], vs[b])
            vs[a], vs[b] = lo, hi
        qg = row0 + lanes
        for i in range(NSEL):
            c = vs[i]
            chunkids[i] = c
            rowvec = (((c // 8) % 8) * 16 + c // 64) * 8192 + qg
            flat = lanes * NSEL + i
            plsc.store_scatter(idxlist, [flat // 128, flat % 128], rowvec)
            plsc.store_scatter(coff, [splat_i(i), lanes], (c % 8) * 16)

        for j in range(5):
            pltpu.async_copy(d_hbm.at[idxlist.at[j]],
                             cand.at[pl.ds(j * 128, 128)], gsem).wait()

        def r1_body(i, c2):
            co = plsc.load_gather(coff, [splat_i(i), lanes])
            acc = _INF
            for t in range(CH):
                acc = jnp.minimum(
                    acc, plsc.load_gather(cand, [lanes * NSEL + i, co + t]))
            plsc.store_scatter(r1, [splat_i(i), lanes], acc)
            return c2

        lax.fori_loop(0, NSEL, r1_body, 0)
        for i in range(NSEL, 48):
            r1[i] = _INF
        for u in range(3):
            acc = _INF
            for t in range(CH):
                acc = jnp.minimum(acc, r1[16 * u + t])
            r2[u] = acc

        def p4_body(it, c2):
            v = jnp.minimum(jnp.minimum(r2[0], r2[1]), r2[2])
            eu = jnp.where(r2[0] == v, splat_i(0),
                           jnp.where(r2[1] == v, splat_i(1), splat_i(2)))
            baseu = eu * 16
            ei = baseu
            for t in range(15, -1, -1):
                gvals = plsc.load_gather(r1, [baseu + t, lanes])
                ei = jnp.where(gvals == v, baseu + t, ei)
            crow = lanes * NSEL + ei
            co = plsc.load_gather(coff, [ei, lanes])
            et = splat_i(0)
            for t in range(15, -1, -1):
                gvals = plsc.load_gather(cand, [crow, co + t])
                et = jnp.where(gvals == v, splat_i(t), et)
            chunk = plsc.load_gather(chunkids, [ei, lanes])
            oidx = chunk * CH + et
            plsc.store_scatter(valstage, [lanes, splat_i(it)], v)
            plsc.store_scatter(idxstage, [lanes, splat_i(it)], oidx)
            plsc.store_scatter(cand, [crow, co + et], _INF)
            acc = _INF
            for t in range(CH):
                acc = jnp.minimum(
                    acc, plsc.load_gather(cand, [crow, co + t]))
            plsc.store_scatter(r1, [ei, lanes], acc)
            acc = _INF
            for t in range(CH):
                acc = jnp.minimum(
                    acc, plsc.load_gather(r1, [baseu + t, lanes]))
            plsc.store_scatter(r2, [eu, lanes], acc)
            return c2

        lax.fori_loop(0, K, p4_body, 0)

        pltpu.sync_copy(valstage, vals_hbm.at[pl.ds(row0, 16)])
        pltpu.sync_copy(idxstage, idx_hbm.at[pl.ds(row0, 16)])
        return carry

    lax.fori_loop(0, GPW, group_body, 0)


def _sc_stage(d, m1):
    mesh = plsc.VectorSubcoreMesh(
        core_axis_name="c", subcore_axis_name="s",
        num_cores=2, num_subcores=16)
    kfun = functools.partial(
        pl.kernel, _sc_body, mesh=mesh,
        compiler_params=pltpu.CompilerParams(
            use_tc_tiling_on_sc=False, needs_layout_passes=False),
        out_type=[
            jax.ShapeDtypeStruct((B * S, K), F32),
            jax.ShapeDtypeStruct((B * S, K), I32),
        ],
        scratch_types=[
            pltpu.VMEM((NCH, 16), F32),
            pltpu.VMEM((64, 16), F32),
            pltpu.VMEM((4, 16), F32),
            pltpu.VMEM((NSEL * 16, 128), F32),
            pltpu.VMEM((NSEL, 16), I32),
            pltpu.VMEM((NSEL, 16), I32),
            pltpu.VMEM((5, 128), I32),
            pltpu.VMEM((48, 16), F32),
            pltpu.VMEM((3, 16), F32),
            pltpu.VMEM((16, K), F32),
            pltpu.VMEM((16, K), I32),
            pltpu.SemaphoreType.DMA,
        ],
    )()
    d_view = d.reshape((NBLK // 128) * (N // NBLK) * B * S, 128)
    m1_view = m1.reshape(B * NCH, S)
    return kfun(d_view, m1_view)


def kernel(xyz, new_xyz):
    d, m1 = _tc_stage(xyz, new_xyz)
    vals, idx = _sc_stage(d, m1)
    return (vals.reshape(B, S, K), idx.reshape(B, S, K))


# speedup vs baseline: 1.2679x; 1.2679x over previous
import functools

import jax
import jax.numpy as jnp
from jax import lax
from jax.experimental import pallas as pl
from jax.experimental.pallas import tpu as pltpu
from jax.experimental.pallas import tpu_sc as plsc

K = 32
CH = 16
B, S, N, C = 8, 1024, 16384, 64
NCH = N // CH
NBLK = 1024
NW = 32
GROUPS = (B * S) // CH
GPW = GROUPS // NW
F32 = jnp.float32
I32 = jnp.int32


def _batcher_pairs(n):
    pairs = []
    p = 1
    while p < n:
        k = p
        while k >= 1:
            for j in range(k % p, n - k, 2 * k):
                for i in range(0, k):
                    if (i + j) // (2 * p) == (i + j + k) // (2 * p):
                        pairs.append((i + j, i + j + k))
            k //= 2
        p *= 2
    return pairs


NSEL = 40
_SORT64 = _batcher_pairs(64)


SBLK = 1024


def _tc_body(q_ref, x_ref, d_ref, m1_ref):
    q = q_ref[0]
    x = x_ref[0]
    qn = jnp.sum(q * q, axis=-1)
    xn = jnp.sum(x * x, axis=-1)
    ab = lax.dot_general(q, x, (((1,), (1,)), ((), ())),
                         preferred_element_type=F32)
    d = (-2.0 * ab + qn[:, None]) + xn[None, :]
    for j in range(NBLK // 128):
        d_ref[j, 0] = d[:, j * 128:(j + 1) * 128]
    ba = lax.dot_general(x, q, (((1,), (1,)), ((), ())),
                         preferred_element_type=F32)
    dt = (-2.0 * ba + qn[None, :]) + xn[:, None]
    m1t = jnp.min(dt.reshape(NBLK // CH, CH, SBLK), axis=1)
    for j in range(SBLK // 128):
        m1_ref[0, :, j] = m1t[:, j * 128:(j + 1) * 128]


def _tc_stage(xyz, new_xyz):
    nb = xyz.shape[0]
    grid = (nb, N // NBLK)
    return pl.pallas_call(
        _tc_body,
        grid=grid,
        in_specs=[
            pl.BlockSpec((1, SBLK, C), lambda b, n: (b, 0, 0)),
            pl.BlockSpec((1, NBLK, C), lambda b, n: (b, n, 0)),
        ],
        out_specs=[
            pl.BlockSpec((NBLK // 128, 1, SBLK, 128),
                         lambda b, n: (0, n, b, 0)),
            pl.BlockSpec((1, NBLK // CH, S // 128, 128),
                         lambda b, n: (b, n, 0, 0)),
        ],
        out_shape=[
            jax.ShapeDtypeStruct((NBLK // 128, N // NBLK, nb * S, 128), F32),
            jax.ShapeDtypeStruct((nb, NCH, S // 128, 128), F32),
        ],
    )(new_xyz, xyz)


def _make_sc_body(nb):
    gpw = (nb * S // CH) // NW
    return functools.partial(_sc_body_impl, nb, gpw)


def _sc_body_impl(nb, gpw, d_hbm, m1_hbm, vals_hbm, idx_hbm,
                  m1buf, l2, l3, cand, chunkids, coff, idxlist, r1, r2,
                  valstage, idxstage, gsem):
    lanes = jnp.arange(16, dtype=I32)
    _INF = jnp.full((16,), jnp.inf, F32)
    wid = lax.axis_index("s") * 2 + lax.axis_index("c")

    def splat_i(v):
        return jnp.full((16,), v, I32)

    def group_body(g, carry):
        gid = wid * gpw + g
        row0 = gid * CH
        bb = gid // (S // CH)
        qq = (gid % (S // CH)) * CH
        pltpu.sync_copy(m1_hbm.at[pl.ds(bb * NCH, NCH), pl.ds(qq, 16)],
                        m1buf)

        def l2_body(j, c2):
            acc = _INF
            for t in range(CH):
                acc = jnp.minimum(
                    acc, plsc.load_gather(m1buf, [splat_i(j * 16 + t), lanes]))
            plsc.store_scatter(l2, [splat_i(j), lanes], acc)
            return c2

        lax.fori_loop(0, 64, l2_body, 0)
        for i in range(4):
            acc = _INF
            for t in range(CH):
                acc = jnp.minimum(acc, l2[16 * i + t])
            l3[i] = acc

        def p2_body(it, c2):
            v = jnp.minimum(jnp.minimum(l3[0], l3[1]),
                            jnp.minimum(l3[2], l3[3]))
            e3 = splat_i(3)
            for i in (2, 1, 0):
                e3 = jnp.where(l3[i] == v, splat_i(i), e3)
            base2 = e3 * 16
            e2 = base2
            for t in range(15, -1, -1):
                gvals = plsc.load_gather(l2, [base2 + t, lanes])
                e2 = jnp.where(gvals == v, base2 + t, e2)
            base1 = e2 * 16
            e1 = base1
            for t in range(15, -1, -1):
                gvals = plsc.load_gather(m1buf, [base1 + t, lanes])
                e1 = jnp.where(gvals == v, base1 + t, e1)
            plsc.store_scatter(chunkids, [splat_i(it), lanes], e1)
            plsc.store_scatter(m1buf, [e1, lanes], _INF)
            acc = _INF
            for t in range(CH):
                acc = jnp.minimum(acc, plsc.load_gather(m1buf, [base1 + t, lanes]))
            plsc.store_scatter(l2, [e2, lanes], acc)
            acc = _INF
            for t in range(CH):
                acc = jnp.minimum(acc, plsc.load_gather(l2, [e3 * 16 + t, lanes]))
            plsc.store_scatter(l3, [e3, lanes], acc)
            return c2

        lax.fori_loop(0, NSEL, p2_body, 0)

        vs = [chunkids[i] for i in range(NSEL)]
        vs += [jnp.full((16,), NCH, I32) for _ in range(64 - NSEL)]
        for (a, b) in _SORT64:
            lo = jnp.minimum(vs[a], vs[b])
            hi = jnp.maximum(vs[a], vs[b])
            vs[a], vs[b] = lo, hi
        qg = row0 + lanes
        for i in range(NSEL):
            c = vs[i]
            chunkids[i] = c
            rowvec = (((c // 8) % 8) * 16 + c // 64) * (nb * S) + qg
            flat = lanes * NSEL + i
            plsc.store_scatter(idxlist, [flat // 128, flat % 128], rowvec)
            plsc.store_scatter(coff, [splat_i(i), lanes], (c % 8) * 16)

        for j in range(5):
            pltpu.async_copy(d_hbm.at[idxlist.at[j]],
                             cand.at[pl.ds(j * 128, 128)], gsem).wait()

        def r1_body(i, c2):
            co = plsc.load_gather(coff, [splat_i(i), lanes])
            acc = _INF
            for t in range(CH):
                acc = jnp.minimum(
                    acc, plsc.load_gather(cand, [lanes * NSEL + i, co + t]))
            plsc.store_scatter(r1, [splat_i(i), lanes], acc)
            return c2

        lax.fori_loop(0, NSEL, r1_body, 0)
        for i in range(NSEL, 48):
            r1[i] = _INF
        for u in range(3):
            acc = _INF
            for t in range(CH):
                acc = jnp.minimum(acc, r1[16 * u + t])
            r2[u] = acc

        def p4_body(it, c2):
            v = jnp.minimum(jnp.minimum(r2[0], r2[1]), r2[2])
            eu = jnp.where(r2[0] == v, splat_i(0),
                           jnp.where(r2[1] == v, splat_i(1), splat_i(2)))
            baseu = eu * 16
            ei = baseu
            for t in range(15, -1, -1):
                gvals = plsc.load_gather(r1, [baseu + t, lanes])
                ei = jnp.where(gvals == v, baseu + t, ei)
            crow = lanes * NSEL + ei
            co = plsc.load_gather(coff, [ei, lanes])
            et = splat_i(0)
            for t in range(15, -1, -1):
                gvals = plsc.load_gather(cand, [crow, co + t])
                et = jnp.where(gvals == v, splat_i(t), et)
            chunk = plsc.load_gather(chunkids, [ei, lanes])
            oidx = chunk * CH + et
            plsc.store_scatter(valstage, [lanes, splat_i(it)], v)
            plsc.store_scatter(idxstage, [lanes, splat_i(it)], oidx)
            plsc.store_scatter(cand, [crow, co + et], _INF)
            acc = _INF
            for t in range(CH):
                acc = jnp.minimum(
                    acc, plsc.load_gather(cand, [crow, co + t]))
            plsc.store_scatter(r1, [ei, lanes], acc)
            acc = _INF
            for t in range(CH):
                acc = jnp.minimum(
                    acc, plsc.load_gather(r1, [baseu + t, lanes]))
            plsc.store_scatter(r2, [eu, lanes], acc)
            return c2

        lax.fori_loop(0, K, p4_body, 0)

        pltpu.sync_copy(valstage, vals_hbm.at[pl.ds(row0, 16)])
        pltpu.sync_copy(idxstage, idx_hbm.at[pl.ds(row0, 16)])
        return carry

    lax.fori_loop(0, gpw, group_body, 0)


def _sc_stage(d, m1):
    nb = m1.shape[0]
    mesh = plsc.VectorSubcoreMesh(
        core_axis_name="c", subcore_axis_name="s",
        num_cores=2, num_subcores=16)
    kfun = functools.partial(
        pl.kernel, _make_sc_body(nb), mesh=mesh,
        compiler_params=pltpu.CompilerParams(
            use_tc_tiling_on_sc=False, needs_layout_passes=False),
        out_type=[
            jax.ShapeDtypeStruct((nb * S, K), F32),
            jax.ShapeDtypeStruct((nb * S, K), I32),
        ],
        scratch_types=[
            pltpu.VMEM((NCH, 16), F32),
            pltpu.VMEM((64, 16), F32),
            pltpu.VMEM((4, 16), F32),
            pltpu.VMEM((NSEL * 16, 128), F32),
            pltpu.VMEM((NSEL, 16), I32),
            pltpu.VMEM((NSEL, 16), I32),
            pltpu.VMEM((5, 128), I32),
            pltpu.VMEM((48, 16), F32),
            pltpu.VMEM((3, 16), F32),
            pltpu.VMEM((16, K), F32),
            pltpu.VMEM((16, K), I32),
            pltpu.SemaphoreType.DMA,
        ],
    )()
    d_view = d.reshape((NBLK // 128) * (N // NBLK) * nb * S, 128)
    m1_view = m1.reshape(nb * NCH, S)
    return kfun(d_view, m1_view)


NB_CHAIN = 2


def kernel(xyz, new_xyz):
    vals_parts, idx_parts = [], []
    for ci in range(B // NB_CHAIN):
        sl = slice(ci * NB_CHAIN, (ci + 1) * NB_CHAIN)
        d, m1 = _tc_stage(xyz[sl], new_xyz[sl])
        vals, idx = _sc_stage(d, m1)
        vals_parts.append(vals.reshape(NB_CHAIN, S, K))
        idx_parts.append(idx.reshape(NB_CHAIN, S, K))
    return (jnp.concatenate(vals_parts), jnp.concatenate(idx_parts))
